# baseline (speedup 1.0000x reference)
import functools

import jax
import jax.numpy as jnp
from jax import lax
from jax.experimental import pallas as pl
from jax.experimental.pallas import tpu as pltpu
from jax.experimental.pallas import tpu_sc as plsc

_C = 128
_NW = 32


def _gather_add(mem, feat, idx):
    n, d = mem.shape
    b = idx.shape[0]
    n_chunks = -(-b // _C)
    last_base = b - _C
    q, r = divmod(n_chunks, _NW)
    npairs = -(-(q + 1) // 2)

    mesh = plsc.VectorSubcoreMesh(core_axis_name="c", subcore_axis_name="s")

    @functools.partial(
        pl.kernel,
        mesh=mesh,
        out_type=jax.ShapeDtypeStruct((b, d), jnp.float32),
        scratch_types=[
            pltpu.VMEM((_C,), jnp.int32),
            pltpu.VMEM((_C,), jnp.int32),
            pltpu.VMEM((_C, d), jnp.float32),
            pltpu.VMEM((_C, d), jnp.float32),
            pltpu.SemaphoreType.DMA,
            pltpu.SemaphoreType.DMA,
        ],
    )
    def k(mem_hbm, feat_hbm, idx_hbm, out_hbm,
          idx0, idx1, bm0, bm1, ga0, ga1):
        wid = lax.axis_index("s") * 2 + lax.axis_index("c")
        nmine = q + (wid < r)
        s0 = wid * q + jnp.minimum(wid, r)

        ring = ((idx0, bm0, ga0), (idx1, bm1, ga1))

        def chunk_base(lc):
            return jnp.minimum((s0 + lc) * _C, last_base)

        def prefetch(lc, bufs):
            idxv, bm, ga = bufs
            base = chunk_base(lc)
            pltpu.sync_copy(idx_hbm.at[pl.ds(base, _C)], idxv)
            pltpu.async_copy(mem_hbm.at[idxv], bm, ga)

        def step(lc, bufs):
            idxv, bm, ga = bufs
            pltpu.make_async_copy(mem_hbm.at[idxv], bm, ga).wait()
            pltpu.async_copy(feat_hbm.at[idxv], bm, ga, add=True).wait()
            pltpu.sync_copy(bm, out_hbm.at[pl.ds(chunk_base(lc), _C)])

            @pl.when(lc + 2 < nmine)
            def _():
                prefetch(lc + 2, bufs)

        @pl.when(0 < nmine)
        def _():
            prefetch(0, ring[0])

        @pl.when(1 < nmine)
        def _():
            prefetch(1, ring[1])

        def pair_body(g, carry):
            for half in range(2):
                lc = 2 * g + half

                @pl.when(lc < nmine)
                def _():
                    step(lc, ring[half])

            return carry

        lax.fori_loop(0, npairs, pair_body, 0)

    return k(mem, feat, idx)


def kernel(memory, node_features, time_w, time_b, timestamps, source_nodes, n_layers):
    del time_w, time_b, timestamps, n_layers
    return _gather_add(memory, node_features, source_nodes.astype(jnp.int32))


# speedup vs baseline: 1.2430x; 1.2430x over previous
import functools

import jax
import jax.numpy as jnp
from jax import lax
from jax.experimental import pallas as pl
from jax.experimental.pallas import tpu as pltpu
from jax.experimental.pallas import tpu_sc as plsc

_C = 128
_NW = 32
_R = 3


def _gather_add(mem, feat, idx):
    n, d = mem.shape
    b = idx.shape[0]
    n_chunks = -(-b // _C)
    last_base = b - _C
    q, r = divmod(n_chunks, _NW)
    ntriples = -(-(q + 1) // _R)

    mesh = plsc.VectorSubcoreMesh(core_axis_name="c", subcore_axis_name="s")

    @functools.partial(
        pl.kernel,
        mesh=mesh,
        out_type=jax.ShapeDtypeStruct((b, d), jnp.float32),
        scratch_types=(
            [pltpu.VMEM((_C,), jnp.int32)] * _R
            + [pltpu.VMEM((_C, d), jnp.float32)] * _R
            + [pltpu.SemaphoreType.DMA] * (2 * _R)
        ),
    )
    def k(mem_hbm, feat_hbm, idx_hbm, out_hbm, *scratch):
        idxv = scratch[0:_R]
        bm = scratch[_R:2 * _R]
        ga = scratch[2 * _R:3 * _R]
        st = scratch[3 * _R:4 * _R]

        wid = lax.axis_index("s") * 2 + lax.axis_index("c")
        nmine = q + (wid < r)
        s0 = wid * q + jnp.minimum(wid, r)

        def chunk_base(lc):
            return jnp.minimum((s0 + lc) * _C, last_base)

        def issue_a(lc, j):
            pltpu.sync_copy(idx_hbm.at[pl.ds(chunk_base(lc), _C)], idxv[j])
            pltpu.async_copy(mem_hbm.at[idxv[j]], bm[j], ga[j])

        def wait_ga(j):
            pltpu.make_async_copy(mem_hbm.at[idxv[j]], bm[j], ga[j]).wait()

        def issue_b(j):
            pltpu.async_copy(feat_hbm.at[idxv[j]], bm[j], ga[j], add=True)

        def wait_st(j):
            pltpu.make_async_copy(bm[j], out_hbm.at[pl.ds(0, _C)], st[j]).wait()

        @pl.when(0 < nmine)
        def _():
            issue_a(0, 0)

        @pl.when(1 < nmine)
        def _():
            issue_a(1, 1)

        @pl.when(0 < nmine)
        def _():
            wait_ga(0)
            issue_b(0)

        def triple_body(g, carry):
            for j in range(_R):
                i = g * _R + j
                j1, j2 = (j + 1) % _R, (j + 2) % _R

                @pl.when(i < nmine)
                def _():
                    wait_ga(j)
                    pltpu.async_copy(bm[j], out_hbm.at[pl.ds(chunk_base(i), _C)],
                                     st[j])

                @pl.when(i + 1 < nmine)
                def _():
                    wait_ga(j1)
                    issue_b(j1)

                @pl.when(i + 2 < nmine)
                def _():
                    @pl.when(i >= 1)
                    def _():
                        wait_st(j2)
                    issue_a(i + 2, j2)

            return carry

        lax.fori_loop(0, ntriples, triple_body, 0)

        for j in range(_R):
            @pl.when(j < nmine)
            def _():
                wait_st(j)

    return k(mem, feat, idx)


def kernel(memory, node_features, time_w, time_b, timestamps, source_nodes, n_layers):
    del time_w, time_b, timestamps, n_layers
    return _gather_add(memory, node_features, source_nodes.astype(jnp.int32))


# speedup vs baseline: 1.3629x; 1.0965x over previous
import functools

import jax
import jax.numpy as jnp
from jax import lax
from jax.experimental import pallas as pl
from jax.experimental.pallas import tpu as pltpu
from jax.experimental.pallas import tpu_sc as plsc

_C = 128
_NW = 32
_R = 5


def _gather_add(mem, feat, idx):
    n, d = mem.shape
    b = idx.shape[0]
    n_chunks = -(-b // _C)
    last_base = b - _C
    q, r = divmod(n_chunks, _NW)
    ngroups = -(-(q + 1) // _R)

    mesh = plsc.VectorSubcoreMesh(core_axis_name="c", subcore_axis_name="s")

    @functools.partial(
        pl.kernel,
        mesh=mesh,
        out_type=jax.ShapeDtypeStruct((b, d), jnp.float32),
        scratch_types=(
            [pltpu.VMEM((_C,), jnp.int32)] * _R
            + [pltpu.VMEM((_C, d), jnp.float32)] * _R
            + [pltpu.SemaphoreType.DMA] * (3 * _R)
        ),
    )
    def k(mem_hbm, feat_hbm, idx_hbm, out_hbm, *scratch):
        idxv = scratch[0:_R]
        bm = scratch[_R:2 * _R]
        ga = scratch[2 * _R:3 * _R]
        st = scratch[3 * _R:4 * _R]
        ix = scratch[4 * _R:5 * _R]

        wid = lax.axis_index("s") * 2 + lax.axis_index("c")
        nmine = q + (wid < r)
        s0 = wid * q + jnp.minimum(wid, r)

        def chunk_base(lc):
            return jnp.minimum((s0 + lc) * _C, last_base)

        def issue_idx(lc, j):
            pltpu.async_copy(idx_hbm.at[pl.ds(chunk_base(lc), _C)], idxv[j], ix[j])

        def wait_idx(j):
            pltpu.make_async_copy(idx_hbm.at[pl.ds(0, _C)], idxv[j], ix[j]).wait()

        def issue_a(j):
            pltpu.async_copy(mem_hbm.at[idxv[j]], bm[j], ga[j])

        def wait_ga(j):
            pltpu.make_async_copy(mem_hbm.at[idxv[j]], bm[j], ga[j]).wait()

        def issue_b(j):
            pltpu.async_copy(feat_hbm.at[idxv[j]], bm[j], ga[j], add=True)

        def wait_st(j):
            pltpu.make_async_copy(bm[j], out_hbm.at[pl.ds(0, _C)], st[j]).wait()

        for c in range(_R):
            @pl.when(c < nmine)
            def _():
                issue_idx(c, c)

        for c in range(_R - 1):
            @pl.when(c < nmine)
            def _():
                wait_idx(c)
                issue_a(c)

        for c in range(2):
            @pl.when(c < nmine)
            def _():
                wait_ga(c)
                issue_b(c)

        def group_body(g, carry):
            for j in range(_R):
                i = g * _R + j
                j2 = (j + 2) % _R
                j4 = (j + 4) % _R

                @pl.when(i < nmine)
                def _():
                    wait_ga(j)
                    pltpu.async_copy(bm[j], out_hbm.at[pl.ds(chunk_base(i), _C)],
                                     st[j])

                @pl.when(i + 2 < nmine)
                def _():
                    wait_ga(j2)
                    issue_b(j2)

                @pl.when(i + 4 < nmine)
                def _():
                    @pl.when(i >= 1)
                    def _():
                        wait_st(j4)
                    wait_idx(j4)
                    issue_a(j4)

                @pl.when(i + _R < nmine)
                def _():
                    issue_idx(i + _R, j)

            return carry

        lax.fori_loop(0, ngroups, group_body, 0)

        for j in range(_R):
            @pl.when(j < nmine)
            def _():
                wait_st(j)

    return k(mem, feat, idx)


def kernel(memory, node_features, time_w, time_b, timestamps, source_nodes, n_layers):
    del time_w, time_b, timestamps, n_layers
    return _gather_add(memory, node_features, source_nodes.astype(jnp.int32))
